# baseline (speedup 1.0000x reference)
import functools

import jax
import jax.numpy as jnp
from jax import lax
from jax.experimental import pallas as pl
from jax.experimental.pallas import tpu as pltpu
from jax.experimental.pallas import tpu_sc as plsc

HIDDEN = 1024
ROWS, COLS = 4, 2048
BATCH = ROWS * COLS

_info = plsc.get_sparse_core_info()
NC, NS = _info.num_cores, _info.num_subcores
NW = NC * NS
B_PER_W = BATCH // NW
W_PER_ROW = COLS // B_PER_W

CHUNKS = (8,) * 32
OFFS = tuple(sum(CHUNKS[:i]) for i in range(len(CHUNKS)))
MAXCH = max(CHUNKS)
NBUF = 12

_mesh = plsc.VectorSubcoreMesh(core_axis_name="c", subcore_axis_name="s")


@functools.partial(
    pl.kernel,
    mesh=_mesh,
    out_type=jax.ShapeDtypeStruct((ROWS, COLS, HIDDEN), jnp.float32),
    scratch_types=(
        [pltpu.VMEM((B_PER_W,), jnp.int32)]
        + [pltpu.VMEM((MAXCH, HIDDEN), jnp.float32)] * NBUF
        + [pltpu.SemaphoreType.DMA] * (2 * NBUF)
    ),
)
def _sc_gather(idx_hbm, table_hbm, out_hbm, idx_v, *rest):
    bufs = rest[:NBUF]
    gsems = rest[NBUF:2 * NBUF]
    osems = rest[2 * NBUF:]
    wid = lax.axis_index("s") * NC + lax.axis_index("c")
    row = wid // W_PER_ROW
    col = (wid % W_PER_ROW) * B_PER_W

    pltpu.sync_copy(idx_hbm.at[row, pl.ds(col, B_PER_W)], idx_v)

    def start_gather(g):
        b = g % NBUF
        ch = CHUNKS[g]
        return pltpu.async_copy(
            table_hbm.at[idx_v.at[pl.ds(OFFS[g], ch)]],
            bufs[b].at[pl.ds(0, ch)], gsems[b])

    nch = len(CHUNKS)
    gat = [None] * NBUF
    outs = [None] * NBUF
    for j in range(min(NBUF, nch)):
        gat[j] = start_gather(j)
    sched = None
    for g in range(nch):
        b = g % NBUF
        if sched is not None:
            sb, sg = sched
            outs[sb].wait()
            gat[sb] = start_gather(sg)
            outs[sb] = None
            sched = None
        gat[b].wait()
        ch = CHUNKS[g]
        outs[b] = pltpu.async_copy(
            bufs[b].at[pl.ds(0, ch)],
            out_hbm.at[row, pl.ds(col + OFFS[g], ch)], osems[b])
        if g + NBUF < nch:
            sched = (b, g + NBUF)
    for o in outs:
        if o is not None:
            o.wait()


def kernel(input_ids, word_embeddings):
    return _sc_gather(input_ids.astype(jnp.int32), word_embeddings)


# speedup vs baseline: 1.0392x; 1.0392x over previous
import functools

import jax
import jax.numpy as jnp
from jax import lax
from jax.experimental import pallas as pl
from jax.experimental.pallas import tpu as pltpu
from jax.experimental.pallas import tpu_sc as plsc

HIDDEN = 1024
ROWS, COLS = 4, 2048
BATCH = ROWS * COLS

_info = plsc.get_sparse_core_info()
NC, NS = _info.num_cores, _info.num_subcores
NW = NC * NS
B_PER_W = BATCH // NW
W_PER_ROW = COLS // B_PER_W

CHUNKS = (16,) * 16
OFFS = tuple(sum(CHUNKS[:i]) for i in range(len(CHUNKS)))
MAXCH = max(CHUNKS)
NBUF = 6

_mesh = plsc.VectorSubcoreMesh(core_axis_name="c", subcore_axis_name="s")


@functools.partial(
    pl.kernel,
    mesh=_mesh,
    out_type=jax.ShapeDtypeStruct((ROWS, COLS, HIDDEN), jnp.float32),
    scratch_types=(
        [pltpu.VMEM((B_PER_W,), jnp.int32)]
        + [pltpu.VMEM((MAXCH, HIDDEN), jnp.float32)] * NBUF
        + [pltpu.SemaphoreType.DMA] * (2 * NBUF)
    ),
)
def _sc_gather(idx_hbm, table_hbm, out_hbm, idx_v, *rest):
    bufs = rest[:NBUF]
    gsems = rest[NBUF:2 * NBUF]
    osems = rest[2 * NBUF:]
    wid = lax.axis_index("s") * NC + lax.axis_index("c")
    row = wid // W_PER_ROW
    col = (wid % W_PER_ROW) * B_PER_W

    pltpu.sync_copy(idx_hbm.at[row, pl.ds(col, B_PER_W)], idx_v)

    def start_gather(g):
        b = g % NBUF
        ch = CHUNKS[g]
        return pltpu.async_copy(
            table_hbm.at[idx_v.at[pl.ds(OFFS[g], ch)]],
            bufs[b].at[pl.ds(0, ch)], gsems[b])

    nch = len(CHUNKS)
    gat = [None] * NBUF
    outs = [None] * NBUF
    for j in range(min(NBUF, nch)):
        gat[j] = start_gather(j)
    sched = None
    for g in range(nch):
        b = g % NBUF
        if sched is not None:
            sb, sg = sched
            outs[sb].wait()
            gat[sb] = start_gather(sg)
            outs[sb] = None
            sched = None
        gat[b].wait()
        ch = CHUNKS[g]
        outs[b] = pltpu.async_copy(
            bufs[b].at[pl.ds(0, ch)],
            out_hbm.at[row, pl.ds(col + OFFS[g], ch)], osems[b])
        if g + NBUF < nch:
            sched = (b, g + NBUF)
    for o in outs:
        if o is not None:
            o.wait()


def kernel(input_ids, word_embeddings):
    return _sc_gather(input_ids.astype(jnp.int32), word_embeddings)
